# Initial kernel scaffold; baseline (speedup 1.0000x reference)
import jax
import jax.numpy as jnp
from jax.experimental import pallas as pl


def kernel(x_loc, x_evt, edge_index, W_loc, b_loc, W_evt, b_evt, W_l, b_l, W_r, W1, b1, W2, b2):
    raise NotImplementedError("write your pallas kernel here")


# speedup vs baseline: 7.3563x; 7.3563x over previous
import functools

import jax
import jax.numpy as jnp
from jax import lax
from jax.experimental import pallas as pl
from jax.experimental.pallas import tpu as pltpu
from jax.experimental.pallas import tpu_sc as plsc

N_LOC = 10000
N_EVT = 10000
E = 320000
D = 128
H = 256
HH = 128
RB = 400
GRID = N_LOC // RB
EB = 128
EPAD = 320512
NB = EPAD // EB
G = 8
NQ = NB // G
NTILES = 16
ACC_ROWS = 10008
CNT_PAD = 10112
RPT = 624
MAXJ = (NQ + NTILES - 1) // NTILES


def _tc_proj_body(xe_ref, we_ref, be_ref, xl_ref, wl_ref, bl_ref,
                  evt_ref, loc_ref):
    ye = jnp.dot(xe_ref[...], we_ref[...], preferred_element_type=jnp.float32)
    ye = jnp.maximum(ye + be_ref[...], 0.0)
    evt_ref[0] = ye[:, :HH]
    evt_ref[1] = ye[:, HH:]
    yl = jnp.dot(xl_ref[...], wl_ref[...], preferred_element_type=jnp.float32)
    loc_ref[...] = jnp.maximum(yl + bl_ref[...], 0.0)


_tc_proj = pl.pallas_call(
    _tc_proj_body,
    grid=(GRID,),
    in_specs=[
        pl.BlockSpec((RB, D), lambda i: (i, 0)),
        pl.BlockSpec((D, H), lambda i: (0, 0)),
        pl.BlockSpec((1, H), lambda i: (0, 0)),
        pl.BlockSpec((RB, D), lambda i: (i, 0)),
        pl.BlockSpec((D, H), lambda i: (0, 0)),
        pl.BlockSpec((1, H), lambda i: (0, 0)),
    ],
    out_specs=[
        pl.BlockSpec((2, RB, HH), lambda i: (0, i, 0)),
        pl.BlockSpec((RB, H), lambda i: (i, 0)),
    ],
    out_shape=[
        jax.ShapeDtypeStruct((2, N_EVT, HH), jnp.float32),
        jax.ShapeDtypeStruct((N_LOC, H), jnp.float32),
    ],
)


def _sc_body(evt_hbm, srcA_hbm, srcB_hbm, dst_hbm, agg0_hbm, agg1_hbm,
             cnt_hbm, rows_v, sidx_v, didx_v, zbuf_v, zc_v, ones_v,
             acc_sh, cnt_sh, sem):
    c = lax.axis_index("c")
    s = lax.axis_index("s")

    zero16 = jnp.zeros((16,), jnp.float32)
    one16 = jnp.ones((16,), jnp.float32)
    for k in range(EB // 16):
        ones_v[pl.ds(k * 16, 16)] = one16

    def _zb(i, carry):
        for k in range(HH // 16):
            zbuf_v[i, pl.ds(k * 16, 16)] = zero16
        return carry
    lax.fori_loop(0, 64, _zb, 0)

    def _zc(i, carry):
        zc_v[pl.ds(i * 16, 16)] = zero16
        return carry
    lax.fori_loop(0, 125, _zc, 0)

    r0 = s * RPT
    for k in range(9):
        pltpu.sync_copy(zbuf_v, acc_sh.at[pl.ds(r0 + k * 64, 64)])
    pltpu.sync_copy(zbuf_v.at[pl.ds(0, 48)], acc_sh.at[pl.ds(r0 + 576, 48)])

    @pl.when(s == NTILES - 1)
    def _():
        pltpu.sync_copy(zbuf_v.at[pl.ds(0, ACC_ROWS - NTILES * RPT)],
                        acc_sh.at[pl.ds(NTILES * RPT, ACC_ROWS - NTILES * RPT)])

    @pl.when(jnp.logical_and(c == 0, s < 5))
    def _():
        pltpu.sync_copy(zc_v, cnt_sh.at[pl.ds(s * 2000, 2000)])

    @pl.when(jnp.logical_and(c == 0, s == 5))
    def _():
        pltpu.sync_copy(zc_v.at[pl.ds(0, CNT_PAD - N_LOC)],
                        cnt_sh.at[pl.ds(N_LOC, CNT_PAD - N_LOC)])

    plsc.subcore_barrier()

    def body(j, carry):
        q = s + NTILES * j

        @pl.when(q < NQ)
        def _():
            row0 = q * G

            @pl.when(c == 0)
            def _():
                pltpu.sync_copy(srcA_hbm.at[pl.ds(row0, G)], sidx_v)

            @pl.when(c == 1)
            def _():
                pltpu.sync_copy(srcB_hbm.at[pl.ds(row0, G)], sidx_v)

            pltpu.sync_copy(dst_hbm.at[pl.ds(row0, G)], didx_v)

            for w in range(4):
                cps = [
                    pltpu.async_copy(evt_hbm.at[sidx_v.at[w * 2 + g]],
                                     rows_v.at[pl.ds(g * EB, EB)], sem)
                    for g in range(2)
                ]
                for g in range(2):
                    cps[g].wait()
                    pltpu.sync_copy(rows_v.at[pl.ds(g * EB, EB)],
                                    acc_sh.at[didx_v.at[w * 2 + g]], add=True)

                @pl.when(c == 0)
                def _():
                    for g in range(2):
                        pltpu.sync_copy(ones_v,
                                        cnt_sh.at[didx_v.at[w * 2 + g]],
                                        add=True)

        return carry

    lax.fori_loop(0, MAXJ, body, 0)

    plsc.subcore_barrier()

    @pl.when(c == 0)
    def _():
        pltpu.sync_copy(acc_sh.at[pl.ds(r0, RPT)], agg0_hbm.at[pl.ds(r0, RPT)])

        @pl.when(s == NTILES - 1)
        def _():
            pltpu.sync_copy(acc_sh.at[pl.ds(NTILES * RPT, N_LOC - NTILES * RPT)],
                            agg0_hbm.at[pl.ds(NTILES * RPT, N_LOC - NTILES * RPT)])

        @pl.when(s == 0)
        def _():
            pltpu.sync_copy(cnt_sh, cnt_hbm)

    @pl.when(c == 1)
    def _():
        pltpu.sync_copy(acc_sh.at[pl.ds(r0, RPT)], agg1_hbm.at[pl.ds(r0, RPT)])

        @pl.when(s == NTILES - 1)
        def _():
            pltpu.sync_copy(acc_sh.at[pl.ds(NTILES * RPT, N_LOC - NTILES * RPT)],
                            agg1_hbm.at[pl.ds(NTILES * RPT, N_LOC - NTILES * RPT)])


_sc_agg = functools.partial(
    pl.kernel,
    out_type=[
        jax.ShapeDtypeStruct((N_LOC, HH), jnp.float32),
        jax.ShapeDtypeStruct((N_LOC, HH), jnp.float32),
        jax.ShapeDtypeStruct((CNT_PAD,), jnp.float32),
    ],
    mesh=plsc.VectorSubcoreMesh(core_axis_name="c", subcore_axis_name="s"),
    scratch_types=[
        pltpu.VMEM((2 * EB, HH), jnp.float32),
        pltpu.VMEM((G, EB), jnp.int32),
        pltpu.VMEM((G, EB), jnp.int32),
        pltpu.VMEM((64, HH), jnp.float32),
        pltpu.VMEM((2000,), jnp.float32),
        pltpu.VMEM((EB,), jnp.float32),
        pltpu.VMEM_SHARED((ACC_ROWS, HH), jnp.float32),
        pltpu.VMEM_SHARED((CNT_PAD,), jnp.float32),
        pltpu.SemaphoreType.DMA,
    ],
)(_sc_body)


def _tc_head_body(a0_ref, a1_ref, cnt_ref, loc_ref, wl_ref, bl_ref, wr_ref,
                  w1_ref, b1_ref, w2_ref, b2_ref, out_ref):
    inv = 1.0 / jnp.maximum(cnt_ref[0], 1.0)
    m0 = a0_ref[...] * inv
    m1 = a1_ref[...] * inv
    wl = wl_ref[...]
    conv = (jnp.dot(m0, wl[:HH], preferred_element_type=jnp.float32)
            + jnp.dot(m1, wl[HH:], preferred_element_type=jnp.float32)
            + jnp.dot(loc_ref[...], wr_ref[...],
                      preferred_element_type=jnp.float32)
            + bl_ref[...])
    lh = jnp.maximum(conv, 0.0)
    h = jnp.dot(lh, w1_ref[...], preferred_element_type=jnp.float32)
    h = jnp.maximum(h + b1_ref[...], 0.0)
    lg = jnp.sum(h * w2_ref[...], axis=1, keepdims=True) + b2_ref[...]
    out_ref[0] = lg


_tc_head = pl.pallas_call(
    _tc_head_body,
    grid=(GRID,),
    in_specs=[
        pl.BlockSpec((RB, HH), lambda i: (i, 0)),
        pl.BlockSpec((RB, HH), lambda i: (i, 0)),
        pl.BlockSpec((1, RB, 1), lambda i: (i, 0, 0)),
        pl.BlockSpec((RB, H), lambda i: (i, 0)),
        pl.BlockSpec((H, H), lambda i: (0, 0)),
        pl.BlockSpec((1, H), lambda i: (0, 0)),
        pl.BlockSpec((H, H), lambda i: (0, 0)),
        pl.BlockSpec((H, HH), lambda i: (0, 0)),
        pl.BlockSpec((1, HH), lambda i: (0, 0)),
        pl.BlockSpec((1, HH), lambda i: (0, 0)),
        pl.BlockSpec((1, 1), lambda i: (0, 0)),
    ],
    out_specs=[pl.BlockSpec((1, RB, 1), lambda i: (i, 0, 0))],
    out_shape=[jax.ShapeDtypeStruct((GRID, RB, 1), jnp.float32)],
)


def kernel(x_loc, x_evt, edge_index, W_loc, b_loc, W_evt, b_evt,
           W_l, b_l, W_r, W1, b1, W2, b2):
    pad = EPAD - E
    src = jnp.concatenate(
        [edge_index[0], jnp.zeros((pad,), jnp.int32)]).reshape(NB, EB)
    dst = jnp.concatenate(
        [edge_index[1], jnp.full((pad,), N_LOC, jnp.int32)]).reshape(NB, EB)
    srcB = src + N_EVT

    evt_pair, loc_x = _tc_proj(x_evt, W_evt, b_evt.reshape(1, H),
                               x_loc, W_loc, b_loc.reshape(1, H))
    evt_flat = evt_pair.reshape(2 * N_EVT, HH)

    agg0, agg1, cnt = _sc_agg(evt_flat, src, srcB, dst)

    (out3,) = _tc_head(agg0, agg1, cnt[:N_LOC].reshape(GRID, RB, 1), loc_x,
                       W_l, b_l.reshape(1, H), W_r, W1, b1.reshape(1, HH),
                       W2.reshape(1, HH), b2.reshape(1, 1))
    return out3.reshape(N_LOC)


# speedup vs baseline: 7.3662x; 1.0013x over previous
import functools

import jax
import jax.numpy as jnp
from jax import lax
from jax.experimental import pallas as pl
from jax.experimental.pallas import tpu as pltpu
from jax.experimental.pallas import tpu_sc as plsc

N_LOC = 10000
N_EVT = 10000
E = 320000
D = 128
H = 256
HH = 128
RB = 400
GRID = N_LOC // RB
EB = 128
EPAD = 320512
NB = EPAD // EB
G = 8
NQ = NB // G
NTILES = 16
ACC_ROWS = 10008
CNT_PAD = 10112
RPT = 624
MAXJ = (NQ + NTILES - 1) // NTILES


def _tc_proj_body(xe_ref, we_ref, be_ref, xl_ref, wl_ref, bl_ref,
                  evt_ref, loc_ref):
    ye = jnp.dot(xe_ref[...], we_ref[...], preferred_element_type=jnp.float32)
    ye = jnp.maximum(ye + be_ref[...], 0.0)
    evt_ref[0] = ye[:, :HH]
    evt_ref[1] = ye[:, HH:]
    yl = jnp.dot(xl_ref[...], wl_ref[...], preferred_element_type=jnp.float32)
    loc_ref[...] = jnp.maximum(yl + bl_ref[...], 0.0)


_tc_proj = pl.pallas_call(
    _tc_proj_body,
    grid=(GRID,),
    in_specs=[
        pl.BlockSpec((RB, D), lambda i: (i, 0)),
        pl.BlockSpec((D, H), lambda i: (0, 0)),
        pl.BlockSpec((1, H), lambda i: (0, 0)),
        pl.BlockSpec((RB, D), lambda i: (i, 0)),
        pl.BlockSpec((D, H), lambda i: (0, 0)),
        pl.BlockSpec((1, H), lambda i: (0, 0)),
    ],
    out_specs=[
        pl.BlockSpec((2, RB, HH), lambda i: (0, i, 0)),
        pl.BlockSpec((RB, H), lambda i: (i, 0)),
    ],
    out_shape=[
        jax.ShapeDtypeStruct((2, N_EVT, HH), jnp.float32),
        jax.ShapeDtypeStruct((N_LOC, H), jnp.float32),
    ],
)


def _sc_body(evt_hbm, srcA_hbm, srcB_hbm, dst_hbm, agg0_hbm, agg1_hbm,
             cntA_hbm, cntB_hbm, rows_v, sidx_v, didx_v, zbuf_v, zc_v, ones_v,
             acc_sh, cnt_sh, sem_g, sem_s0, sem_s1, sem_c):
    c = lax.axis_index("c")
    s = lax.axis_index("s")

    zero16 = jnp.zeros((16,), jnp.float32)
    one16 = jnp.ones((16,), jnp.float32)
    for k in range(EB // 16):
        ones_v[pl.ds(k * 16, 16)] = one16

    def _zb(i, carry):
        for k in range(HH // 16):
            zbuf_v[i, pl.ds(k * 16, 16)] = zero16
        return carry
    lax.fori_loop(0, 64, _zb, 0)

    def _zc(i, carry):
        zc_v[pl.ds(i * 16, 16)] = zero16
        return carry
    lax.fori_loop(0, 125, _zc, 0)

    r0 = s * RPT
    for k in range(9):
        pltpu.sync_copy(zbuf_v, acc_sh.at[pl.ds(r0 + k * 64, 64)])
    pltpu.sync_copy(zbuf_v.at[pl.ds(0, 48)], acc_sh.at[pl.ds(r0 + 576, 48)])

    @pl.when(s == NTILES - 1)
    def _():
        pltpu.sync_copy(zbuf_v.at[pl.ds(0, ACC_ROWS - NTILES * RPT)],
                        acc_sh.at[pl.ds(NTILES * RPT, ACC_ROWS - NTILES * RPT)])

    @pl.when(s < 5)
    def _():
        pltpu.sync_copy(zc_v, cnt_sh.at[pl.ds(s * 2000, 2000)])

    @pl.when(s == 5)
    def _():
        pltpu.sync_copy(zc_v.at[pl.ds(0, CNT_PAD - N_LOC)],
                        cnt_sh.at[pl.ds(N_LOC, CNT_PAD - N_LOC)])

    plsc.subcore_barrier()

    def body(j, carry):
        q = s + NTILES * j

        @pl.when(q < NQ)
        def _():
            row0 = q * G

            @pl.when(c == 0)
            def _():
                pltpu.sync_copy(srcA_hbm.at[pl.ds(row0, G)], sidx_v)

            @pl.when(c == 1)
            def _():
                pltpu.sync_copy(srcB_hbm.at[pl.ds(row0, G)], sidx_v)

            pltpu.sync_copy(dst_hbm.at[pl.ds(row0, G)], didx_v)

            half = NQ // 2
            do_cnt = jnp.logical_or(jnp.logical_and(c == 0, q < half),
                                    jnp.logical_and(c == 1, q >= half))
            sem_s = [sem_s0, sem_s1]

            def buf(b):
                return rows_v.at[pl.ds((b % 2) * EB, EB)]

            gathers = [pltpu.async_copy(evt_hbm.at[sidx_v.at[0]], buf(0),
                                        sem_g)]
            scatters = []
            cnts = []
            for b in range(G):
                gathers[b].wait()
                scatters.append(
                    pltpu.async_copy(buf(b), acc_sh.at[didx_v.at[b]],
                                     sem_s[b % 2], add=True))

                @pl.when(do_cnt)
                def _(b=b):
                    cnts.append(
                        pltpu.async_copy(ones_v, cnt_sh.at[didx_v.at[b]],
                                         sem_c, add=True))

                if b + 1 < G:
                    if b >= 1:
                        scatters[b - 1].wait()
                    gathers.append(
                        pltpu.async_copy(evt_hbm.at[sidx_v.at[b + 1]],
                                         buf(b + 1), sem_g))
            scatters[G - 2].wait()
            scatters[G - 1].wait()

            @pl.when(do_cnt)
            def _():
                for cp in cnts:
                    cp.wait()

        return carry

    lax.fori_loop(0, MAXJ, body, 0)

    plsc.subcore_barrier()

    @pl.when(c == 0)
    def _():
        pltpu.sync_copy(acc_sh.at[pl.ds(r0, RPT)], agg0_hbm.at[pl.ds(r0, RPT)])

        @pl.when(s == NTILES - 1)
        def _():
            pltpu.sync_copy(acc_sh.at[pl.ds(NTILES * RPT, N_LOC - NTILES * RPT)],
                            agg0_hbm.at[pl.ds(NTILES * RPT, N_LOC - NTILES * RPT)])

        @pl.when(s == 0)
        def _():
            pltpu.sync_copy(cnt_sh, cntA_hbm)

    @pl.when(c == 1)
    def _():
        pltpu.sync_copy(acc_sh.at[pl.ds(r0, RPT)], agg1_hbm.at[pl.ds(r0, RPT)])

        @pl.when(s == NTILES - 1)
        def _():
            pltpu.sync_copy(acc_sh.at[pl.ds(NTILES * RPT, N_LOC - NTILES * RPT)],
                            agg1_hbm.at[pl.ds(NTILES * RPT, N_LOC - NTILES * RPT)])

        @pl.when(s == 0)
        def _():
            pltpu.sync_copy(cnt_sh, cntB_hbm)


_sc_agg = functools.partial(
    pl.kernel,
    out_type=[
        jax.ShapeDtypeStruct((N_LOC, HH), jnp.float32),
        jax.ShapeDtypeStruct((N_LOC, HH), jnp.float32),
        jax.ShapeDtypeStruct((CNT_PAD,), jnp.float32),
        jax.ShapeDtypeStruct((CNT_PAD,), jnp.float32),
    ],
    mesh=plsc.VectorSubcoreMesh(core_axis_name="c", subcore_axis_name="s"),
    scratch_types=[
        pltpu.VMEM((2 * EB, HH), jnp.float32),
        pltpu.VMEM((G, EB), jnp.int32),
        pltpu.VMEM((G, EB), jnp.int32),
        pltpu.VMEM((64, HH), jnp.float32),
        pltpu.VMEM((2000,), jnp.float32),
        pltpu.VMEM((EB,), jnp.float32),
        pltpu.VMEM_SHARED((ACC_ROWS, HH), jnp.float32),
        pltpu.VMEM_SHARED((CNT_PAD,), jnp.float32),
        pltpu.SemaphoreType.DMA,
        pltpu.SemaphoreType.DMA,
        pltpu.SemaphoreType.DMA,
        pltpu.SemaphoreType.DMA,
    ],
)(_sc_body)


def _tc_head_body(a0_ref, a1_ref, cntA_ref, cntB_ref, loc_ref, wl_ref,
                  bl_ref, wr_ref, w1_ref, b1_ref, w2_ref, b2_ref, out_ref):
    cnt = cntA_ref[0] + cntB_ref[0]
    inv = 1.0 / jnp.maximum(cnt, 1.0)
    m0 = a0_ref[...] * inv
    m1 = a1_ref[...] * inv
    wl = wl_ref[...]
    conv = (jnp.dot(m0, wl[:HH], preferred_element_type=jnp.float32)
            + jnp.dot(m1, wl[HH:], preferred_element_type=jnp.float32)
            + jnp.dot(loc_ref[...], wr_ref[...],
                      preferred_element_type=jnp.float32)
            + bl_ref[...])
    lh = jnp.maximum(conv, 0.0)
    h = jnp.dot(lh, w1_ref[...], preferred_element_type=jnp.float32)
    h = jnp.maximum(h + b1_ref[...], 0.0)
    lg = jnp.sum(h * w2_ref[...], axis=1, keepdims=True) + b2_ref[...]
    out_ref[0] = lg


_tc_head = pl.pallas_call(
    _tc_head_body,
    grid=(GRID,),
    in_specs=[
        pl.BlockSpec((RB, HH), lambda i: (i, 0)),
        pl.BlockSpec((RB, HH), lambda i: (i, 0)),
        pl.BlockSpec((1, RB, 1), lambda i: (i, 0, 0)),
        pl.BlockSpec((1, RB, 1), lambda i: (i, 0, 0)),
        pl.BlockSpec((RB, H), lambda i: (i, 0)),
        pl.BlockSpec((H, H), lambda i: (0, 0)),
        pl.BlockSpec((1, H), lambda i: (0, 0)),
        pl.BlockSpec((H, H), lambda i: (0, 0)),
        pl.BlockSpec((H, HH), lambda i: (0, 0)),
        pl.BlockSpec((1, HH), lambda i: (0, 0)),
        pl.BlockSpec((1, HH), lambda i: (0, 0)),
        pl.BlockSpec((1, 1), lambda i: (0, 0)),
    ],
    out_specs=[pl.BlockSpec((1, RB, 1), lambda i: (i, 0, 0))],
    out_shape=[jax.ShapeDtypeStruct((GRID, RB, 1), jnp.float32)],
)


def kernel(x_loc, x_evt, edge_index, W_loc, b_loc, W_evt, b_evt,
           W_l, b_l, W_r, W1, b1, W2, b2):
    pad = EPAD - E
    src = jnp.concatenate(
        [edge_index[0], jnp.zeros((pad,), jnp.int32)]).reshape(NB, EB)
    dst = jnp.concatenate(
        [edge_index[1], jnp.full((pad,), N_LOC, jnp.int32)]).reshape(NB, EB)
    srcB = src + N_EVT

    evt_pair, loc_x = _tc_proj(x_evt, W_evt, b_evt.reshape(1, H),
                               x_loc, W_loc, b_loc.reshape(1, H))
    evt_flat = evt_pair.reshape(2 * N_EVT, HH)

    agg0, agg1, cntA, cntB = _sc_agg(evt_flat, src, srcB, dst)

    (out3,) = _tc_head(agg0, agg1, cntA[:N_LOC].reshape(GRID, RB, 1),
                       cntB[:N_LOC].reshape(GRID, RB, 1), loc_x,
                       W_l, b_l.reshape(1, H), W_r, W1, b1.reshape(1, HH),
                       W2.reshape(1, HH), b2.reshape(1, 1))
    return out3.reshape(N_LOC)


# speedup vs baseline: 7.8267x; 1.0625x over previous
import functools

import jax
import jax.numpy as jnp
from jax import lax
from jax.experimental import pallas as pl
from jax.experimental.pallas import tpu as pltpu
from jax.experimental.pallas import tpu_sc as plsc

N_LOC = 10000
N_EVT = 10000
E = 320000
D = 128
H = 256
HH = 128
RB = 2000
GRID = N_LOC // RB
EB = 128
EPAD = 320512
NB = EPAD // EB
G = 8
NQ = NB // G
NTILES = 16
ACC_ROWS = 10008
CNT_PAD = 10112
RPT = 624
MAXJ = (NQ + NTILES - 1) // NTILES


def _tc_proj_body(xe_ref, we_ref, be_ref, xl_ref, wl_ref, bl_ref,
                  evt_ref, loc_ref):
    ye = jnp.dot(xe_ref[...], we_ref[...], preferred_element_type=jnp.float32)
    ye = jnp.maximum(ye + be_ref[...], 0.0)
    evt_ref[0] = ye[:, :HH]
    evt_ref[1] = ye[:, HH:]
    yl = jnp.dot(xl_ref[...], wl_ref[...], preferred_element_type=jnp.float32)
    loc_ref[...] = jnp.maximum(yl + bl_ref[...], 0.0)


_tc_proj = pl.pallas_call(
    _tc_proj_body,
    grid=(GRID,),
    in_specs=[
        pl.BlockSpec((RB, D), lambda i: (i, 0)),
        pl.BlockSpec((D, H), lambda i: (0, 0)),
        pl.BlockSpec((1, H), lambda i: (0, 0)),
        pl.BlockSpec((RB, D), lambda i: (i, 0)),
        pl.BlockSpec((D, H), lambda i: (0, 0)),
        pl.BlockSpec((1, H), lambda i: (0, 0)),
    ],
    out_specs=[
        pl.BlockSpec((2, RB, HH), lambda i: (0, i, 0)),
        pl.BlockSpec((RB, H), lambda i: (i, 0)),
    ],
    out_shape=[
        jax.ShapeDtypeStruct((2, N_EVT, HH), jnp.float32),
        jax.ShapeDtypeStruct((N_LOC, H), jnp.float32),
    ],
)


def _sc_body(evt_hbm, srcA_hbm, srcB_hbm, dst_hbm, agg0_hbm, agg1_hbm,
             cntA_hbm, cntB_hbm, rows_v, sidx_v, didx_v, zbuf_v, zc_v, ones_v,
             acc_sh, cnt_sh, sem_g, sem_s0, sem_s1, sem_c):
    c = lax.axis_index("c")
    s = lax.axis_index("s")

    zero16 = jnp.zeros((16,), jnp.float32)
    one16 = jnp.ones((16,), jnp.float32)
    for k in range(EB // 16):
        ones_v[pl.ds(k * 16, 16)] = one16

    def _zb(i, carry):
        for k in range(HH // 16):
            zbuf_v[i, pl.ds(k * 16, 16)] = zero16
        return carry
    lax.fori_loop(0, 64, _zb, 0)

    def _zc(i, carry):
        zc_v[pl.ds(i * 16, 16)] = zero16
        return carry
    lax.fori_loop(0, 125, _zc, 0)

    r0 = s * RPT
    for k in range(9):
        pltpu.sync_copy(zbuf_v, acc_sh.at[pl.ds(r0 + k * 64, 64)])
    pltpu.sync_copy(zbuf_v.at[pl.ds(0, 48)], acc_sh.at[pl.ds(r0 + 576, 48)])

    @pl.when(s == NTILES - 1)
    def _():
        pltpu.sync_copy(zbuf_v.at[pl.ds(0, ACC_ROWS - NTILES * RPT)],
                        acc_sh.at[pl.ds(NTILES * RPT, ACC_ROWS - NTILES * RPT)])

    @pl.when(s < 5)
    def _():
        pltpu.sync_copy(zc_v, cnt_sh.at[pl.ds(s * 2000, 2000)])

    @pl.when(s == 5)
    def _():
        pltpu.sync_copy(zc_v.at[pl.ds(0, CNT_PAD - N_LOC)],
                        cnt_sh.at[pl.ds(N_LOC, CNT_PAD - N_LOC)])

    plsc.subcore_barrier()

    def body(j, carry):
        q = s + NTILES * j

        @pl.when(q < NQ)
        def _():
            row0 = q * G

            @pl.when(c == 0)
            def _():
                pltpu.sync_copy(srcA_hbm.at[pl.ds(row0, G)], sidx_v)

            @pl.when(c == 1)
            def _():
                pltpu.sync_copy(srcB_hbm.at[pl.ds(row0, G)], sidx_v)

            pltpu.sync_copy(dst_hbm.at[pl.ds(row0, G)], didx_v)

            half = NQ // 2
            do_cnt = jnp.logical_or(jnp.logical_and(c == 0, q < half),
                                    jnp.logical_and(c == 1, q >= half))
            sem_s = [sem_s0, sem_s1]

            def buf(b):
                return rows_v.at[pl.ds((b % 2) * EB, EB)]

            gathers = [pltpu.async_copy(evt_hbm.at[sidx_v.at[0]], buf(0),
                                        sem_g)]
            scatters = []
            cnts = []
            for b in range(G):
                gathers[b].wait()
                scatters.append(
                    pltpu.async_copy(buf(b), acc_sh.at[didx_v.at[b]],
                                     sem_s[b % 2], add=True))

                @pl.when(do_cnt)
                def _(b=b):
                    cnts.append(
                        pltpu.async_copy(ones_v, cnt_sh.at[didx_v.at[b]],
                                         sem_c, add=True))

                if b + 1 < G:
                    if b >= 1:
                        scatters[b - 1].wait()
                    gathers.append(
                        pltpu.async_copy(evt_hbm.at[sidx_v.at[b + 1]],
                                         buf(b + 1), sem_g))
            scatters[G - 2].wait()
            scatters[G - 1].wait()

            @pl.when(do_cnt)
            def _():
                for cp in cnts:
                    cp.wait()

        return carry

    lax.fori_loop(0, MAXJ, body, 0)

    plsc.subcore_barrier()

    @pl.when(c == 0)
    def _():
        pltpu.sync_copy(acc_sh.at[pl.ds(r0, RPT)], agg0_hbm.at[pl.ds(r0, RPT)])

        @pl.when(s == NTILES - 1)
        def _():
            pltpu.sync_copy(acc_sh.at[pl.ds(NTILES * RPT, N_LOC - NTILES * RPT)],
                            agg0_hbm.at[pl.ds(NTILES * RPT, N_LOC - NTILES * RPT)])

        @pl.when(s == 0)
        def _():
            pltpu.sync_copy(cnt_sh, cntA_hbm)

    @pl.when(c == 1)
    def _():
        pltpu.sync_copy(acc_sh.at[pl.ds(r0, RPT)], agg1_hbm.at[pl.ds(r0, RPT)])

        @pl.when(s == NTILES - 1)
        def _():
            pltpu.sync_copy(acc_sh.at[pl.ds(NTILES * RPT, N_LOC - NTILES * RPT)],
                            agg1_hbm.at[pl.ds(NTILES * RPT, N_LOC - NTILES * RPT)])

        @pl.when(s == 0)
        def _():
            pltpu.sync_copy(cnt_sh, cntB_hbm)


_sc_agg = functools.partial(
    pl.kernel,
    out_type=[
        jax.ShapeDtypeStruct((N_LOC, HH), jnp.float32),
        jax.ShapeDtypeStruct((N_LOC, HH), jnp.float32),
        jax.ShapeDtypeStruct((CNT_PAD,), jnp.float32),
        jax.ShapeDtypeStruct((CNT_PAD,), jnp.float32),
    ],
    mesh=plsc.VectorSubcoreMesh(core_axis_name="c", subcore_axis_name="s"),
    scratch_types=[
        pltpu.VMEM((2 * EB, HH), jnp.float32),
        pltpu.VMEM((G, EB), jnp.int32),
        pltpu.VMEM((G, EB), jnp.int32),
        pltpu.VMEM((64, HH), jnp.float32),
        pltpu.VMEM((2000,), jnp.float32),
        pltpu.VMEM((EB,), jnp.float32),
        pltpu.VMEM_SHARED((ACC_ROWS, HH), jnp.float32),
        pltpu.VMEM_SHARED((CNT_PAD,), jnp.float32),
        pltpu.SemaphoreType.DMA,
        pltpu.SemaphoreType.DMA,
        pltpu.SemaphoreType.DMA,
        pltpu.SemaphoreType.DMA,
    ],
)(_sc_body)


def _tc_head_body(a0_ref, a1_ref, cntA_ref, cntB_ref, loc_ref, wl_ref,
                  bl_ref, wr_ref, w1_ref, b1_ref, w2_ref, b2_ref, out_ref):
    cnt = cntA_ref[0] + cntB_ref[0]
    inv = 1.0 / jnp.maximum(cnt, 1.0)
    m0 = a0_ref[...] * inv
    m1 = a1_ref[...] * inv
    wl = wl_ref[...]
    conv = (jnp.dot(m0, wl[:HH], preferred_element_type=jnp.float32)
            + jnp.dot(m1, wl[HH:], preferred_element_type=jnp.float32)
            + jnp.dot(loc_ref[...], wr_ref[...],
                      preferred_element_type=jnp.float32)
            + bl_ref[...])
    lh = jnp.maximum(conv, 0.0)
    h = jnp.dot(lh, w1_ref[...], preferred_element_type=jnp.float32)
    h = jnp.maximum(h + b1_ref[...], 0.0)
    lg = jnp.sum(h * w2_ref[...], axis=1, keepdims=True) + b2_ref[...]
    out_ref[0] = lg


_tc_head = pl.pallas_call(
    _tc_head_body,
    grid=(GRID,),
    in_specs=[
        pl.BlockSpec((RB, HH), lambda i: (i, 0)),
        pl.BlockSpec((RB, HH), lambda i: (i, 0)),
        pl.BlockSpec((1, RB, 1), lambda i: (i, 0, 0)),
        pl.BlockSpec((1, RB, 1), lambda i: (i, 0, 0)),
        pl.BlockSpec((RB, H), lambda i: (i, 0)),
        pl.BlockSpec((H, H), lambda i: (0, 0)),
        pl.BlockSpec((1, H), lambda i: (0, 0)),
        pl.BlockSpec((H, H), lambda i: (0, 0)),
        pl.BlockSpec((H, HH), lambda i: (0, 0)),
        pl.BlockSpec((1, HH), lambda i: (0, 0)),
        pl.BlockSpec((1, HH), lambda i: (0, 0)),
        pl.BlockSpec((1, 1), lambda i: (0, 0)),
    ],
    out_specs=[pl.BlockSpec((1, RB, 1), lambda i: (i, 0, 0))],
    out_shape=[jax.ShapeDtypeStruct((GRID, RB, 1), jnp.float32)],
)


def kernel(x_loc, x_evt, edge_index, W_loc, b_loc, W_evt, b_evt,
           W_l, b_l, W_r, W1, b1, W2, b2):
    pad = EPAD - E
    src = jnp.concatenate(
        [edge_index[0], jnp.zeros((pad,), jnp.int32)]).reshape(NB, EB)
    dst = jnp.concatenate(
        [edge_index[1], jnp.full((pad,), N_LOC, jnp.int32)]).reshape(NB, EB)
    srcB = src + N_EVT

    evt_pair, loc_x = _tc_proj(x_evt, W_evt, b_evt.reshape(1, H),
                               x_loc, W_loc, b_loc.reshape(1, H))
    evt_flat = evt_pair.reshape(2 * N_EVT, HH)

    agg0, agg1, cntA, cntB = _sc_agg(evt_flat, src, srcB, dst)

    (out3,) = _tc_head(agg0, agg1, cntA[:N_LOC].reshape(GRID, RB, 1),
                       cntB[:N_LOC].reshape(GRID, RB, 1), loc_x,
                       W_l, b_l.reshape(1, H), W_r, W1, b1.reshape(1, HH),
                       W2.reshape(1, HH), b2.reshape(1, 1))
    return out3.reshape(N_LOC)


# speedup vs baseline: 7.8432x; 1.0021x over previous
import functools

import jax
import jax.numpy as jnp
from jax import lax
from jax.experimental import pallas as pl
from jax.experimental.pallas import tpu as pltpu
from jax.experimental.pallas import tpu_sc as plsc

N_LOC = 10000
N_EVT = 10000
E = 320000
D = 128
H = 256
HH = 128
RB = 2000
GRID = N_LOC // RB
EB = 128
EPAD = 320512
NB = EPAD // EB
G = 8
NQ = NB // G
NTILES = 16
ACC_ROWS = 10008
CNT_PAD = 10112
RPT = 624
MAXJ = (NQ + NTILES - 1) // NTILES


def _tc_evt_body(xe_ref, we_ref, be_ref, evt_ref):
    ye = jnp.dot(xe_ref[...], we_ref[...], preferred_element_type=jnp.float32)
    ye = jnp.maximum(ye + be_ref[...], 0.0)
    evt_ref[0] = ye[:, :HH]
    evt_ref[1] = ye[:, HH:]


_tc_evt = pl.pallas_call(
    _tc_evt_body,
    grid=(GRID,),
    in_specs=[
        pl.BlockSpec((RB, D), lambda i: (i, 0)),
        pl.BlockSpec((D, H), lambda i: (0, 0)),
        pl.BlockSpec((1, H), lambda i: (0, 0)),
    ],
    out_specs=[pl.BlockSpec((2, RB, HH), lambda i: (0, i, 0))],
    out_shape=[jax.ShapeDtypeStruct((2, N_EVT, HH), jnp.float32)],
)


def _tc_loc_body(xl_ref, wl_ref, bl_ref, loc_ref):
    yl = jnp.dot(xl_ref[...], wl_ref[...], preferred_element_type=jnp.float32)
    loc_ref[...] = jnp.maximum(yl + bl_ref[...], 0.0)


_tc_loc = pl.pallas_call(
    _tc_loc_body,
    grid=(GRID,),
    in_specs=[
        pl.BlockSpec((RB, D), lambda i: (i, 0)),
        pl.BlockSpec((D, H), lambda i: (0, 0)),
        pl.BlockSpec((1, H), lambda i: (0, 0)),
    ],
    out_specs=[pl.BlockSpec((RB, H), lambda i: (i, 0))],
    out_shape=[jax.ShapeDtypeStruct((N_LOC, H), jnp.float32)],
)


def _sc_body(evt_hbm, srcA_hbm, srcB_hbm, dst_hbm, agg0_hbm, agg1_hbm,
             cntA_hbm, cntB_hbm, rows_v, sidx_v, didx_v, zbuf_v, zc_v, ones_v,
             acc_sh, cnt_sh, sem_g, sem_s0, sem_s1, sem_c):
    c = lax.axis_index("c")
    s = lax.axis_index("s")

    zero16 = jnp.zeros((16,), jnp.float32)
    one16 = jnp.ones((16,), jnp.float32)
    for k in range(EB // 16):
        ones_v[pl.ds(k * 16, 16)] = one16

    def _zb(i, carry):
        for k in range(HH // 16):
            zbuf_v[i, pl.ds(k * 16, 16)] = zero16
        return carry
    lax.fori_loop(0, 64, _zb, 0)

    def _zc(i, carry):
        zc_v[pl.ds(i * 16, 16)] = zero16
        return carry
    lax.fori_loop(0, 125, _zc, 0)

    r0 = s * RPT
    for k in range(9):
        pltpu.sync_copy(zbuf_v, acc_sh.at[pl.ds(r0 + k * 64, 64)])
    pltpu.sync_copy(zbuf_v.at[pl.ds(0, 48)], acc_sh.at[pl.ds(r0 + 576, 48)])

    @pl.when(s == NTILES - 1)
    def _():
        pltpu.sync_copy(zbuf_v.at[pl.ds(0, ACC_ROWS - NTILES * RPT)],
                        acc_sh.at[pl.ds(NTILES * RPT, ACC_ROWS - NTILES * RPT)])

    @pl.when(s < 5)
    def _():
        pltpu.sync_copy(zc_v, cnt_sh.at[pl.ds(s * 2000, 2000)])

    @pl.when(s == 5)
    def _():
        pltpu.sync_copy(zc_v.at[pl.ds(0, CNT_PAD - N_LOC)],
                        cnt_sh.at[pl.ds(N_LOC, CNT_PAD - N_LOC)])

    plsc.subcore_barrier()

    def body(j, carry):
        q = s + NTILES * j

        @pl.when(q < NQ)
        def _():
            row0 = q * G

            @pl.when(c == 0)
            def _():
                pltpu.sync_copy(srcA_hbm.at[pl.ds(row0, G)], sidx_v)

            @pl.when(c == 1)
            def _():
                pltpu.sync_copy(srcB_hbm.at[pl.ds(row0, G)], sidx_v)

            pltpu.sync_copy(dst_hbm.at[pl.ds(row0, G)], didx_v)

            half = NQ // 2
            do_cnt = jnp.logical_or(jnp.logical_and(c == 0, q < half),
                                    jnp.logical_and(c == 1, q >= half))
            sem_s = [sem_s0, sem_s1]

            def buf(b):
                return rows_v.at[pl.ds((b % 2) * EB, EB)]

            gathers = [pltpu.async_copy(evt_hbm.at[sidx_v.at[0]], buf(0),
                                        sem_g)]
            scatters = []
            cnts = []
            for b in range(G):
                gathers[b].wait()
                scatters.append(
                    pltpu.async_copy(buf(b), acc_sh.at[didx_v.at[b]],
                                     sem_s[b % 2], add=True))

                @pl.when(do_cnt)
                def _(b=b):
                    cnts.append(
                        pltpu.async_copy(ones_v, cnt_sh.at[didx_v.at[b]],
                                         sem_c, add=True))

                if b + 1 < G:
                    if b >= 1:
                        scatters[b - 1].wait()
                    gathers.append(
                        pltpu.async_copy(evt_hbm.at[sidx_v.at[b + 1]],
                                         buf(b + 1), sem_g))
            scatters[G - 2].wait()
            scatters[G - 1].wait()

            @pl.when(do_cnt)
            def _():
                for cp in cnts:
                    cp.wait()

        return carry

    lax.fori_loop(0, MAXJ, body, 0)

    plsc.subcore_barrier()

    @pl.when(c == 0)
    def _():
        pltpu.sync_copy(acc_sh.at[pl.ds(r0, RPT)], agg0_hbm.at[pl.ds(r0, RPT)])

        @pl.when(s == NTILES - 1)
        def _():
            pltpu.sync_copy(acc_sh.at[pl.ds(NTILES * RPT, N_LOC - NTILES * RPT)],
                            agg0_hbm.at[pl.ds(NTILES * RPT, N_LOC - NTILES * RPT)])

        @pl.when(s == 0)
        def _():
            pltpu.sync_copy(cnt_sh, cntA_hbm)

    @pl.when(c == 1)
    def _():
        pltpu.sync_copy(acc_sh.at[pl.ds(r0, RPT)], agg1_hbm.at[pl.ds(r0, RPT)])

        @pl.when(s == NTILES - 1)
        def _():
            pltpu.sync_copy(acc_sh.at[pl.ds(NTILES * RPT, N_LOC - NTILES * RPT)],
                            agg1_hbm.at[pl.ds(NTILES * RPT, N_LOC - NTILES * RPT)])

        @pl.when(s == 0)
        def _():
            pltpu.sync_copy(cnt_sh, cntB_hbm)


_sc_agg = functools.partial(
    pl.kernel,
    out_type=[
        jax.ShapeDtypeStruct((N_LOC, HH), jnp.float32),
        jax.ShapeDtypeStruct((N_LOC, HH), jnp.float32),
        jax.ShapeDtypeStruct((CNT_PAD,), jnp.float32),
        jax.ShapeDtypeStruct((CNT_PAD,), jnp.float32),
    ],
    mesh=plsc.VectorSubcoreMesh(core_axis_name="c", subcore_axis_name="s"),
    scratch_types=[
        pltpu.VMEM((2 * EB, HH), jnp.float32),
        pltpu.VMEM((G, EB), jnp.int32),
        pltpu.VMEM((G, EB), jnp.int32),
        pltpu.VMEM((64, HH), jnp.float32),
        pltpu.VMEM((2000,), jnp.float32),
        pltpu.VMEM((EB,), jnp.float32),
        pltpu.VMEM_SHARED((ACC_ROWS, HH), jnp.float32),
        pltpu.VMEM_SHARED((CNT_PAD,), jnp.float32),
        pltpu.SemaphoreType.DMA,
        pltpu.SemaphoreType.DMA,
        pltpu.SemaphoreType.DMA,
        pltpu.SemaphoreType.DMA,
    ],
)(_sc_body)


def _tc_head_body(a0_ref, a1_ref, cntA_ref, cntB_ref, loc_ref, wl_ref,
                  bl_ref, wr_ref, w1_ref, b1_ref, w2_ref, b2_ref, out_ref):
    cnt = cntA_ref[0] + cntB_ref[0]
    inv = 1.0 / jnp.maximum(cnt, 1.0)
    m0 = a0_ref[...] * inv
    m1 = a1_ref[...] * inv
    wl = wl_ref[...]
    conv = (jnp.dot(m0, wl[:HH], preferred_element_type=jnp.float32)
            + jnp.dot(m1, wl[HH:], preferred_element_type=jnp.float32)
            + jnp.dot(loc_ref[...], wr_ref[...],
                      preferred_element_type=jnp.float32)
            + bl_ref[...])
    lh = jnp.maximum(conv, 0.0)
    h = jnp.dot(lh, w1_ref[...], preferred_element_type=jnp.float32)
    h = jnp.maximum(h + b1_ref[...], 0.0)
    lg = jnp.sum(h * w2_ref[...], axis=1, keepdims=True) + b2_ref[...]
    out_ref[0] = lg


_tc_head = pl.pallas_call(
    _tc_head_body,
    grid=(GRID,),
    in_specs=[
        pl.BlockSpec((RB, HH), lambda i: (i, 0)),
        pl.BlockSpec((RB, HH), lambda i: (i, 0)),
        pl.BlockSpec((1, RB, 1), lambda i: (i, 0, 0)),
        pl.BlockSpec((1, RB, 1), lambda i: (i, 0, 0)),
        pl.BlockSpec((RB, H), lambda i: (i, 0)),
        pl.BlockSpec((H, H), lambda i: (0, 0)),
        pl.BlockSpec((1, H), lambda i: (0, 0)),
        pl.BlockSpec((H, H), lambda i: (0, 0)),
        pl.BlockSpec((H, HH), lambda i: (0, 0)),
        pl.BlockSpec((1, HH), lambda i: (0, 0)),
        pl.BlockSpec((1, HH), lambda i: (0, 0)),
        pl.BlockSpec((1, 1), lambda i: (0, 0)),
    ],
    out_specs=[pl.BlockSpec((1, RB, 1), lambda i: (i, 0, 0))],
    out_shape=[jax.ShapeDtypeStruct((GRID, RB, 1), jnp.float32)],
)


def kernel(x_loc, x_evt, edge_index, W_loc, b_loc, W_evt, b_evt,
           W_l, b_l, W_r, W1, b1, W2, b2):
    pad = EPAD - E
    src = jnp.concatenate(
        [edge_index[0], jnp.zeros((pad,), jnp.int32)]).reshape(NB, EB)
    dst = jnp.concatenate(
        [edge_index[1], jnp.full((pad,), N_LOC, jnp.int32)]).reshape(NB, EB)
    srcB = src + N_EVT

    (evt_pair,) = _tc_evt(x_evt, W_evt, b_evt.reshape(1, H))
    evt_flat = evt_pair.reshape(2 * N_EVT, HH)

    agg0, agg1, cntA, cntB = _sc_agg(evt_flat, src, srcB, dst)

    (loc_x,) = _tc_loc(x_loc, W_loc, b_loc.reshape(1, H))

    (out3,) = _tc_head(agg0, agg1, cntA[:N_LOC].reshape(GRID, RB, 1),
                       cntB[:N_LOC].reshape(GRID, RB, 1), loc_x,
                       W_l, b_l.reshape(1, H), W_r, W1, b1.reshape(1, HH),
                       W2.reshape(1, HH), b2.reshape(1, 1))
    return out3.reshape(N_LOC)


# speedup vs baseline: 7.8956x; 1.0067x over previous
import functools

import jax
import jax.numpy as jnp
from jax import lax
from jax.experimental import pallas as pl
from jax.experimental.pallas import tpu as pltpu
from jax.experimental.pallas import tpu_sc as plsc

N_LOC = 10000
N_EVT = 10000
E = 320000
D = 128
H = 256
HH = 128
RB = 2000
GRID = N_LOC // RB
EB = 128
EPAD = 320512
NB = EPAD // EB
G = 8
NQ = NB // G
NTILES = 16
ACC_ROWS = 10008
CNT_PAD = 10112
RPT = 624
MAXJ = (NQ + NTILES - 1) // NTILES


def _tc_evt_body(xe_ref, we_ref, be_ref, evt_ref):
    ye = jnp.dot(xe_ref[...], we_ref[...], preferred_element_type=jnp.float32)
    ye = jnp.maximum(ye + be_ref[...], 0.0)
    evt_ref[0] = ye[:, :HH]
    evt_ref[1] = ye[:, HH:]


_tc_evt = pl.pallas_call(
    _tc_evt_body,
    grid=(GRID,),
    in_specs=[
        pl.BlockSpec((RB, D), lambda i: (i, 0)),
        pl.BlockSpec((D, H), lambda i: (0, 0)),
        pl.BlockSpec((1, H), lambda i: (0, 0)),
    ],
    out_specs=[pl.BlockSpec((2, RB, HH), lambda i: (0, i, 0))],
    out_shape=[jax.ShapeDtypeStruct((2, N_EVT, HH), jnp.float32)],
)


def _sc_body(evt_hbm, srcA_hbm, srcB_hbm, dst_hbm, agg0_hbm, agg1_hbm,
             cntA_hbm, cntB_hbm, rows_v, sidx_v, didx_v, zbuf_v, zc_v, ones_v,
             acc_sh, cnt_sh, sem_g, sem_s0, sem_s1, sem_c):
    c = lax.axis_index("c")
    s = lax.axis_index("s")

    zero16 = jnp.zeros((16,), jnp.float32)
    one16 = jnp.ones((16,), jnp.float32)
    for k in range(EB // 16):
        ones_v[pl.ds(k * 16, 16)] = one16

    def _zb(i, carry):
        for k in range(HH // 16):
            zbuf_v[i, pl.ds(k * 16, 16)] = zero16
        return carry
    lax.fori_loop(0, 64, _zb, 0)

    def _zc(i, carry):
        zc_v[pl.ds(i * 16, 16)] = zero16
        return carry
    lax.fori_loop(0, 125, _zc, 0)

    r0 = s * RPT
    for k in range(9):
        pltpu.sync_copy(zbuf_v, acc_sh.at[pl.ds(r0 + k * 64, 64)])
    pltpu.sync_copy(zbuf_v.at[pl.ds(0, 48)], acc_sh.at[pl.ds(r0 + 576, 48)])

    @pl.when(s == NTILES - 1)
    def _():
        pltpu.sync_copy(zbuf_v.at[pl.ds(0, ACC_ROWS - NTILES * RPT)],
                        acc_sh.at[pl.ds(NTILES * RPT, ACC_ROWS - NTILES * RPT)])

    @pl.when(s < 5)
    def _():
        pltpu.sync_copy(zc_v, cnt_sh.at[pl.ds(s * 2000, 2000)])

    @pl.when(s == 5)
    def _():
        pltpu.sync_copy(zc_v.at[pl.ds(0, CNT_PAD - N_LOC)],
                        cnt_sh.at[pl.ds(N_LOC, CNT_PAD - N_LOC)])

    plsc.subcore_barrier()

    def body(j, carry):
        q = s + NTILES * j

        @pl.when(q < NQ)
        def _():
            row0 = q * G

            @pl.when(c == 0)
            def _():
                pltpu.sync_copy(srcA_hbm.at[pl.ds(row0, G)], sidx_v)

            @pl.when(c == 1)
            def _():
                pltpu.sync_copy(srcB_hbm.at[pl.ds(row0, G)], sidx_v)

            pltpu.sync_copy(dst_hbm.at[pl.ds(row0, G)], didx_v)

            half = NQ // 2
            do_cnt = jnp.logical_or(jnp.logical_and(c == 0, q < half),
                                    jnp.logical_and(c == 1, q >= half))
            sem_s = [sem_s0, sem_s1]

            def buf(b):
                return rows_v.at[pl.ds((b % 2) * EB, EB)]

            gathers = [pltpu.async_copy(evt_hbm.at[sidx_v.at[0]], buf(0),
                                        sem_g)]
            scatters = []
            cnts = []
            for b in range(G):
                gathers[b].wait()
                scatters.append(
                    pltpu.async_copy(buf(b), acc_sh.at[didx_v.at[b]],
                                     sem_s[b % 2], add=True))

                @pl.when(do_cnt)
                def _(b=b):
                    cnts.append(
                        pltpu.async_copy(ones_v, cnt_sh.at[didx_v.at[b]],
                                         sem_c, add=True))

                if b + 1 < G:
                    if b >= 1:
                        scatters[b - 1].wait()
                    gathers.append(
                        pltpu.async_copy(evt_hbm.at[sidx_v.at[b + 1]],
                                         buf(b + 1), sem_g))
            scatters[G - 2].wait()
            scatters[G - 1].wait()

            @pl.when(do_cnt)
            def _():
                for cp in cnts:
                    cp.wait()

        return carry

    lax.fori_loop(0, MAXJ, body, 0)

    plsc.subcore_barrier()

    @pl.when(c == 0)
    def _():
        pltpu.sync_copy(acc_sh.at[pl.ds(r0, RPT)], agg0_hbm.at[pl.ds(r0, RPT)])

        @pl.when(s == NTILES - 1)
        def _():
            pltpu.sync_copy(acc_sh.at[pl.ds(NTILES * RPT, N_LOC - NTILES * RPT)],
                            agg0_hbm.at[pl.ds(NTILES * RPT, N_LOC - NTILES * RPT)])

        @pl.when(s == 0)
        def _():
            pltpu.sync_copy(cnt_sh, cntA_hbm)

    @pl.when(c == 1)
    def _():
        pltpu.sync_copy(acc_sh.at[pl.ds(r0, RPT)], agg1_hbm.at[pl.ds(r0, RPT)])

        @pl.when(s == NTILES - 1)
        def _():
            pltpu.sync_copy(acc_sh.at[pl.ds(NTILES * RPT, N_LOC - NTILES * RPT)],
                            agg1_hbm.at[pl.ds(NTILES * RPT, N_LOC - NTILES * RPT)])

        @pl.when(s == 0)
        def _():
            pltpu.sync_copy(cnt_sh, cntB_hbm)


_sc_agg = functools.partial(
    pl.kernel,
    out_type=[
        jax.ShapeDtypeStruct((N_LOC, HH), jnp.float32),
        jax.ShapeDtypeStruct((N_LOC, HH), jnp.float32),
        jax.ShapeDtypeStruct((CNT_PAD,), jnp.float32),
        jax.ShapeDtypeStruct((CNT_PAD,), jnp.float32),
    ],
    mesh=plsc.VectorSubcoreMesh(core_axis_name="c", subcore_axis_name="s"),
    scratch_types=[
        pltpu.VMEM((2 * EB, HH), jnp.float32),
        pltpu.VMEM((G, EB), jnp.int32),
        pltpu.VMEM((G, EB), jnp.int32),
        pltpu.VMEM((64, HH), jnp.float32),
        pltpu.VMEM((2000,), jnp.float32),
        pltpu.VMEM((EB,), jnp.float32),
        pltpu.VMEM_SHARED((ACC_ROWS, HH), jnp.float32),
        pltpu.VMEM_SHARED((CNT_PAD,), jnp.float32),
        pltpu.SemaphoreType.DMA,
        pltpu.SemaphoreType.DMA,
        pltpu.SemaphoreType.DMA,
        pltpu.SemaphoreType.DMA,
    ],
)(_sc_body)


def _tc_head_body(a0_ref, a1_ref, cntA_ref, cntB_ref, xl_ref, wloc_ref,
                  bloc_ref, wl_ref, bl_ref, wr_ref, w1_ref, b1_ref, w2_ref,
                  b2_ref, out_ref):
    yl = jnp.dot(xl_ref[...], wloc_ref[...], preferred_element_type=jnp.float32)
    loc = jnp.maximum(yl + bloc_ref[...], 0.0)
    cnt = cntA_ref[0] + cntB_ref[0]
    inv = 1.0 / jnp.maximum(cnt, 1.0)
    m0 = a0_ref[...] * inv
    m1 = a1_ref[...] * inv
    wl = wl_ref[...]
    conv = (jnp.dot(m0, wl[:HH], preferred_element_type=jnp.float32)
            + jnp.dot(m1, wl[HH:], preferred_element_type=jnp.float32)
            + jnp.dot(loc, wr_ref[...],
                      preferred_element_type=jnp.float32)
            + bl_ref[...])
    lh = jnp.maximum(conv, 0.0)
    h = jnp.dot(lh, w1_ref[...], preferred_element_type=jnp.float32)
    h = jnp.maximum(h + b1_ref[...], 0.0)
    lg = jnp.sum(h * w2_ref[...], axis=1, keepdims=True) + b2_ref[...]
    out_ref[0] = lg


_tc_head = pl.pallas_call(
    _tc_head_body,
    grid=(GRID,),
    in_specs=[
        pl.BlockSpec((RB, HH), lambda i: (i, 0)),
        pl.BlockSpec((RB, HH), lambda i: (i, 0)),
        pl.BlockSpec((1, RB, 1), lambda i: (i, 0, 0)),
        pl.BlockSpec((1, RB, 1), lambda i: (i, 0, 0)),
        pl.BlockSpec((RB, D), lambda i: (i, 0)),
        pl.BlockSpec((D, H), lambda i: (0, 0)),
        pl.BlockSpec((1, H), lambda i: (0, 0)),
        pl.BlockSpec((H, H), lambda i: (0, 0)),
        pl.BlockSpec((1, H), lambda i: (0, 0)),
        pl.BlockSpec((H, H), lambda i: (0, 0)),
        pl.BlockSpec((H, HH), lambda i: (0, 0)),
        pl.BlockSpec((1, HH), lambda i: (0, 0)),
        pl.BlockSpec((1, HH), lambda i: (0, 0)),
        pl.BlockSpec((1, 1), lambda i: (0, 0)),
    ],
    out_specs=[pl.BlockSpec((1, RB, 1), lambda i: (i, 0, 0))],
    out_shape=[jax.ShapeDtypeStruct((GRID, RB, 1), jnp.float32)],
)


def kernel(x_loc, x_evt, edge_index, W_loc, b_loc, W_evt, b_evt,
           W_l, b_l, W_r, W1, b1, W2, b2):
    pad = EPAD - E
    src = jnp.concatenate(
        [edge_index[0], jnp.zeros((pad,), jnp.int32)]).reshape(NB, EB)
    dst = jnp.concatenate(
        [edge_index[1], jnp.full((pad,), N_LOC, jnp.int32)]).reshape(NB, EB)
    srcB = src + N_EVT

    (evt_pair,) = _tc_evt(x_evt, W_evt, b_evt.reshape(1, H))
    evt_flat = evt_pair.reshape(2 * N_EVT, HH)

    agg0, agg1, cntA, cntB = _sc_agg(evt_flat, src, srcB, dst)

    (out3,) = _tc_head(agg0, agg1, cntA[:N_LOC].reshape(GRID, RB, 1),
                       cntB[:N_LOC].reshape(GRID, RB, 1), x_loc, W_loc,
                       b_loc.reshape(1, H),
                       W_l, b_l.reshape(1, H), W_r, W1, b1.reshape(1, HH),
                       W2.reshape(1, HH), b2.reshape(1, 1))
    return out3.reshape(N_LOC)
